# baseline (speedup 1.0000x reference)
import jax
import jax.numpy as jnp
from jax import lax
from jax.experimental import pallas as pl
from jax.experimental.pallas import tpu as pltpu
from jax.experimental.pallas import tpu_sc as plsc

D_MODEL = 64
SCALE = 8.0
NUM_WORKERS = 32
CHUNK = 128


def _emb_body(x_hbm, lut_hbm, out_hbm, idx_v, rows_v, gsem):
    wid = lax.axis_index("s") * 2 + lax.axis_index("c")
    n_chunks = idx_v.shape[0]
    base = wid * (n_chunks * CHUNK)
    pltpu.sync_copy(x_hbm.at[wid], idx_v)

    def chunk_body(i, carry):
        pltpu.async_copy(lut_hbm.at[idx_v.at[i]], rows_v, gsem).wait()

        def scale_row(r, c2):
            for k in range(4):
                rows_v[r, pl.ds(k * 16, 16)] = (
                    rows_v[r, pl.ds(k * 16, 16)] * SCALE
                )
            return c2

        lax.fori_loop(0, CHUNK, scale_row, 0)
        pltpu.sync_copy(rows_v, out_hbm.at[pl.ds(base + i * CHUNK, CHUNK)])
        return carry

    lax.fori_loop(0, n_chunks, chunk_body, 0)


def kernel(x, lut):
    B, S = x.shape
    n = B * S
    idx = x.reshape(n).astype(jnp.int32)
    per_w = n // NUM_WORKERS
    n_chunks = per_w // CHUNK
    idx3 = idx.reshape(NUM_WORKERS, n_chunks, CHUNK)
    mesh = plsc.VectorSubcoreMesh(core_axis_name="c", subcore_axis_name="s")
    out = pl.kernel(
        _emb_body,
        mesh=mesh,
        out_type=jax.ShapeDtypeStruct((n, D_MODEL), jnp.float32),
        scratch_types=[
            pltpu.VMEM((n_chunks, CHUNK), jnp.int32),
            pltpu.VMEM((CHUNK, D_MODEL), jnp.float32),
            pltpu.SemaphoreType.DMA,
        ],
        compiler_params=pltpu.CompilerParams(use_tc_tiling_on_sc=False),
    )(idx3, lut)
    return out.reshape(B, S, D_MODEL)


# speedup vs baseline: 1.2080x; 1.2080x over previous
import jax
import jax.numpy as jnp
from jax import lax
from jax.experimental import pallas as pl
from jax.experimental.pallas import tpu as pltpu
from jax.experimental.pallas import tpu_sc as plsc

D_MODEL = 64
SCALE = 8.0
NUM_WORKERS = 32
CHUNK = 128
NBUF = 4


def _emb_body(x_hbm, lut_hbm, out_hbm, idx_v, bufs, gsems, wsems):
    in_bufs, out_bufs = bufs
    wid = lax.axis_index("s") * 2 + lax.axis_index("c")
    n_chunks = idx_v.shape[0]
    n_groups = n_chunks // NBUF
    base = wid * (n_chunks * CHUNK)
    pltpu.sync_copy(x_hbm.at[wid], idx_v)

    def start_gather(i, b):
        pltpu.async_copy(lut_hbm.at[idx_v.at[i]], in_bufs[b], gsems[b])

    def wait_gather(b):
        pltpu.make_async_copy(lut_hbm.at[idx_v.at[0]], in_bufs[b],
                              gsems[b]).wait()

    def start_write(i, b):
        pltpu.async_copy(out_bufs[b],
                         out_hbm.at[pl.ds(base + i * CHUNK, CHUNK)], wsems[b])

    def wait_write(b):
        pltpu.make_async_copy(out_bufs[b],
                              out_hbm.at[pl.ds(base, CHUNK)], wsems[b]).wait()

    def scale(b):
        def row(r, c):
            for rr in range(4):
                for k in range(4):
                    sl = pl.ds(k * 16, 16)
                    out_bufs[b][r * 4 + rr, sl] = (
                        in_bufs[b][r * 4 + rr, sl] * SCALE
                    )
            return c
        lax.fori_loop(0, CHUNK // 4, row, 0)

    for b in range(NBUF):
        start_gather(b, b)

    for b in range(NBUF):
        wait_gather(b)
        scale(b)
        start_gather(NBUF + b, b)
        start_write(b, b)

    def group(g, carry):
        for b in range(NBUF):
            i = g * NBUF + b
            wait_gather(b)
            wait_write(b)
            scale(b)
            start_gather(i + NBUF, b)
            start_write(i, b)
        return carry

    lax.fori_loop(1, n_groups - 1, group, 0)

    for b in range(NBUF):
        i = (n_groups - 1) * NBUF + b
        wait_gather(b)
        wait_write(b)
        scale(b)
        start_write(i, b)

    for b in range(NBUF):
        wait_write(b)


def kernel(x, lut):
    B, S = x.shape
    n = B * S
    idx = x.reshape(n).astype(jnp.int32)
    per_w = n // NUM_WORKERS
    n_chunks = per_w // CHUNK
    idx3 = idx.reshape(NUM_WORKERS, n_chunks, CHUNK)
    mesh = plsc.VectorSubcoreMesh(core_axis_name="c", subcore_axis_name="s")
    out = pl.kernel(
        _emb_body,
        mesh=mesh,
        out_type=jax.ShapeDtypeStruct((n, D_MODEL), jnp.float32),
        scratch_types=[
            pltpu.VMEM((n_chunks, CHUNK), jnp.int32),
            (
                [pltpu.VMEM((CHUNK, D_MODEL), jnp.float32)
                 for _ in range(NBUF)],
                [pltpu.VMEM((CHUNK, D_MODEL), jnp.float32)
                 for _ in range(NBUF)],
            ),
            [pltpu.SemaphoreType.DMA for _ in range(NBUF)],
            [pltpu.SemaphoreType.DMA for _ in range(NBUF)],
        ],
        compiler_params=pltpu.CompilerParams(use_tc_tiling_on_sc=False),
    )(idx3, lut)
    return out.reshape(B, S, D_MODEL)
